# baseline (speedup 1.0000x reference)
import jax
import jax.numpy as jnp
from jax.experimental import pallas as pl
from jax.experimental.pallas import tpu as pltpu

_BB = 512


def _fused_lenet_kernel(x_ref, w1_ref, w2_ref, wf1_ref, wf2_ref, wf3_ref,
                        o_ref, a1_scr):
    f32 = jnp.float32
    bf16 = jnp.bfloat16
    ones1 = jnp.ones((1, _BB), bf16)
    extra6 = jnp.concatenate([ones1, jnp.zeros((5, _BB), bf16)], axis=0)
    pad16 = jnp.concatenate([ones1, jnp.zeros((15, _BB), bf16)], axis=0)
    extra8 = jnp.concatenate([ones1, jnp.zeros((7, _BB), bf16)], axis=0)

    w1 = w1_ref[...]
    for p in range(12):
        cands = []
        for dh in (0, 1):
            ho = 2 * p + dh
            rows = x_ref[pl.ds(32 * ho, 160), :]
            cands.append(jnp.dot(w1, rows, preferred_element_type=f32))
        m = jnp.maximum(cands[0], cands[1])
        m = jnp.maximum(m[:240], m[240:])
        a1_scr[pl.ds(256 * p, 240), :] = jnp.maximum(m, 0.0).astype(bf16)
        a1_scr[pl.ds(256 * p + 240, 16), :] = pad16

    w2 = w2_ref[...]
    a2_parts = []
    for p in range(5):
        cands = []
        for dh in (0, 1):
            ho = 2 * p + dh
            rows = a1_scr[pl.ds(256 * ho, 752), :]
            cands.append(jnp.dot(w2, rows, preferred_element_type=f32))
        m = jnp.maximum(cands[0], cands[1])
        m = jnp.maximum(m[:256], m[256:])
        a2_parts.append(jnp.maximum(m[:250], 0.0).astype(bf16))

    a2 = jnp.concatenate(a2_parts + [extra6], axis=0)
    h1 = jnp.maximum(
        jnp.dot(wf1_ref[...], a2, preferred_element_type=f32), 0.0).astype(bf16)
    h2 = jnp.maximum(
        jnp.dot(wf2_ref[...], jnp.concatenate([h1, extra8], axis=0),
                preferred_element_type=f32), 0.0).astype(bf16)
    o_ref[...] = jnp.dot(
        wf3_ref[...], jnp.concatenate([h2, extra8], axis=0),
        preferred_element_type=f32)


def _toeplitz_weights(w1p, b1p, w2cat, b2p, wf1r, bf1p, wf2p, bf2p, wf3p, bf3p):
    f32 = jnp.float32
    w1 = w1p[:, :20].reshape(5, 5, 20)
    t1 = jnp.stack([jnp.pad(w1, ((0, 0), (wo, 27 - wo), (0, 0)))
                    for wo in range(24)])
    t1 = t1.transpose(0, 3, 1, 2).reshape(24, 20, 160)
    t1 = jnp.concatenate([t1[0::2], t1[1::2]], axis=0).reshape(480, 160)
    w1t = t1.at[:, 28].set(jnp.tile(b1p[0, :20], 24))

    w2 = w2cat.reshape(9, 128, 128)[:, :20, :50].reshape(3, 3, 20, 50)
    t2 = jnp.stack([jnp.pad(w2, ((0, 0), (wo, 9 - wo), (0, 0), (0, 0)))
                    for wo in range(10)])
    t2 = t2.transpose(0, 4, 1, 2, 3).reshape(10, 50, 3, 240)
    b2c = jnp.tile(b2p[0, :50], 10).reshape(10, 50, 1)
    t2 = jnp.concatenate(
        [t2[:, :, 0], b2c, jnp.zeros((10, 50, 15), f32),
         t2[:, :, 1], jnp.zeros((10, 50, 16), f32), t2[:, :, 2]], axis=2)
    z6 = jnp.zeros((6, 752), f32)
    w2t = jnp.concatenate([t2[0::2].reshape(250, 752), z6,
                           t2[1::2].reshape(250, 752), z6], axis=0)

    wf1 = wf1r.reshape(5, 5, 128, 256)[:, :, :50, :].reshape(1250, 256).T
    wf1t = jnp.concatenate(
        [wf1, bf1p.T, jnp.zeros((256, 5), f32)], axis=1)
    wf2t = jnp.concatenate([wf2p.T, bf2p.T, jnp.zeros((128, 7), f32)], axis=1)
    wf3t = jnp.concatenate([wf3p.T, bf3p.T, jnp.zeros((128, 7), f32)], axis=1)
    bf16 = jnp.bfloat16
    return (w1t.astype(bf16), w2t.astype(bf16), wf1t.astype(bf16),
            wf2t.astype(bf16), wf3t.astype(bf16))


def kernel(w1p, b1p, w2cat, b2p, wf1r, bf1p, wf2p, bf2p, wf3p, bf3p, x):
    B = x.shape[0]
    w1t, w2t, wf1t, wf2t, wf3t = _toeplitz_weights(
        w1p, b1p, w2cat, b2p, wf1r, bf1p, wf2p, bf2p, wf3p, bf3p)
    xt = jnp.concatenate(
        [x.reshape(B, 28, 28), jnp.ones((B, 28, 1), x.dtype),
         jnp.zeros((B, 28, 3), x.dtype)], axis=2)
    xt = xt.reshape(B, 896).T.astype(jnp.bfloat16)

    steps = B // _BB
    logits = pl.pallas_call(
        _fused_lenet_kernel,
        out_shape=jax.ShapeDtypeStruct((128, B), jnp.float32),
        grid=(steps,),
        in_specs=[
            pl.BlockSpec((896, _BB), lambda b: (0, b)),
            pl.BlockSpec((480, 160), lambda b: (0, 0)),
            pl.BlockSpec((512, 752), lambda b: (0, 0)),
            pl.BlockSpec((256, 1256), lambda b: (0, 0)),
            pl.BlockSpec((128, 264), lambda b: (0, 0)),
            pl.BlockSpec((128, 136), lambda b: (0, 0)),
        ],
        out_specs=pl.BlockSpec((128, _BB), lambda b: (0, b)),
        scratch_shapes=[pltpu.VMEM((3072, _BB), jnp.bfloat16)],
        compiler_params=pltpu.CompilerParams(
            dimension_semantics=("parallel",),
            vmem_limit_bytes=64 * 1024 * 1024,
        ),
    )(xt, w1t, w2t, wf1t, wf2t, wf3t)
    return logits[:10, :].T


# speedup vs baseline: 1.0066x; 1.0066x over previous
import jax
import jax.numpy as jnp
from jax.experimental import pallas as pl
from jax.experimental.pallas import tpu as pltpu

_BB = 512


def _fused_lenet_kernel(x_ref, w1_ref, w2_ref, wf1_ref, wf2_ref, wf3_ref,
                        o_ref, a1_scr):
    f32 = jnp.float32
    bf16 = jnp.bfloat16
    ones1 = jnp.ones((1, _BB), bf16)
    extra6 = jnp.concatenate([ones1, jnp.zeros((5, _BB), bf16)], axis=0)
    extra8 = jnp.concatenate([ones1, jnp.zeros((7, _BB), bf16)], axis=0)

    w1 = w1_ref[...]
    for p in range(12):
        cands = []
        for dh in (0, 1):
            ho = 2 * p + dh
            rows = jnp.concatenate(
                [x_ref[pl.ds(32 * ho, 160), :], extra8], axis=0)
            cands.append(jnp.dot(w1, rows, preferred_element_type=f32))
        m = jnp.maximum(cands[0], cands[1])
        m = jnp.maximum(m[:240], m[240:])
        a1_scr[pl.ds(240 * p, 240), :] = jnp.maximum(m, 0.0).astype(bf16)

    w2 = w2_ref[...]
    a2_parts = []
    for p in range(5):
        cands = []
        for dh in (0, 1):
            ho = 2 * p + dh
            rows = jnp.concatenate(
                [a1_scr[pl.ds(240 * ho, 720), :], extra8], axis=0)
            cands.append(jnp.dot(w2, rows, preferred_element_type=f32))
        m = jnp.maximum(cands[0], cands[1])
        m = jnp.maximum(m[:256], m[256:])
        a2_parts.append(jnp.maximum(m[:250], 0.0).astype(bf16))

    a2 = jnp.concatenate(a2_parts + [extra6], axis=0)
    h1 = jnp.maximum(
        jnp.dot(wf1_ref[...], a2, preferred_element_type=f32), 0.0).astype(bf16)
    h2 = jnp.maximum(
        jnp.dot(wf2_ref[...], jnp.concatenate([h1, extra8], axis=0),
                preferred_element_type=f32), 0.0).astype(bf16)
    o_ref[...] = jnp.dot(
        wf3_ref[...], jnp.concatenate([h2, extra8], axis=0),
        preferred_element_type=f32)


def _toeplitz_weights(w1p, b1p, w2cat, b2p, wf1r, bf1p, wf2p, bf2p, wf3p, bf3p):
    f32 = jnp.float32
    w1 = w1p[:, :20].reshape(5, 5, 20)
    t1 = jnp.stack([jnp.pad(w1, ((0, 0), (wo, 27 - wo), (0, 0)))
                    for wo in range(24)])
    t1 = t1.transpose(0, 3, 1, 2).reshape(24, 20, 160)
    t1 = jnp.concatenate([t1[0::2], t1[1::2]], axis=0).reshape(480, 160)
    b1c = jnp.tile(b1p[0, :20], 24).reshape(480, 1)
    w1t = jnp.concatenate([t1, b1c, jnp.zeros((480, 7), f32)], axis=1)

    w2 = w2cat.reshape(9, 128, 128)[:, :20, :50].reshape(3, 3, 20, 50)
    t2 = jnp.stack([jnp.pad(w2, ((0, 0), (wo, 9 - wo), (0, 0), (0, 0)))
                    for wo in range(10)])
    t2 = t2.transpose(0, 4, 1, 2, 3).reshape(10, 50, 720)
    b2c = jnp.tile(b2p[0, :50], 10).reshape(10, 50, 1)
    t2 = jnp.concatenate([t2, b2c, jnp.zeros((10, 50, 7), f32)], axis=2)
    z6 = jnp.zeros((6, 728), f32)
    w2t = jnp.concatenate([t2[0::2].reshape(250, 728), z6,
                           t2[1::2].reshape(250, 728), z6], axis=0)

    wf1 = wf1r.reshape(5, 5, 128, 256)[:, :, :50, :].reshape(1250, 256).T
    wf1t = jnp.concatenate(
        [wf1, bf1p.T, jnp.zeros((256, 5), f32)], axis=1)
    wf2t = jnp.concatenate([wf2p.T, bf2p.T, jnp.zeros((128, 7), f32)], axis=1)
    wf3t = jnp.concatenate([wf3p.T, bf3p.T, jnp.zeros((128, 7), f32)], axis=1)
    bf16 = jnp.bfloat16
    return (w1t.astype(bf16), w2t.astype(bf16), wf1t.astype(bf16),
            wf2t.astype(bf16), wf3t.astype(bf16))


def kernel(w1p, b1p, w2cat, b2p, wf1r, bf1p, wf2p, bf2p, wf3p, bf3p, x):
    B = x.shape[0]
    w1t, w2t, wf1t, wf2t, wf3t = _toeplitz_weights(
        w1p, b1p, w2cat, b2p, wf1r, bf1p, wf2p, bf2p, wf3p, bf3p)
    xt = jnp.pad(x.reshape(B, 28, 28), ((0, 0), (0, 0), (0, 4)))
    xt = xt.reshape(B, 896).T.astype(jnp.bfloat16)

    steps = B // _BB
    logits = pl.pallas_call(
        _fused_lenet_kernel,
        out_shape=jax.ShapeDtypeStruct((128, B), jnp.float32),
        grid=(steps,),
        in_specs=[
            pl.BlockSpec((896, _BB), lambda b: (0, b)),
            pl.BlockSpec((480, 168), lambda b: (0, 0)),
            pl.BlockSpec((512, 728), lambda b: (0, 0)),
            pl.BlockSpec((256, 1256), lambda b: (0, 0)),
            pl.BlockSpec((128, 264), lambda b: (0, 0)),
            pl.BlockSpec((128, 136), lambda b: (0, 0)),
        ],
        out_specs=pl.BlockSpec((128, _BB), lambda b: (0, b)),
        scratch_shapes=[pltpu.VMEM((2880, _BB), jnp.bfloat16)],
        compiler_params=pltpu.CompilerParams(
            dimension_semantics=("parallel",),
            vmem_limit_bytes=64 * 1024 * 1024,
        ),
    )(xt, w1t, w2t, wf1t, wf2t, wf3t)
    return logits[:10, :].T
